# baseline (device time: 83683 ns/iter reference)
import os

import jax
import jax.numpy as jnp
from jax import lax
from jax.experimental import pallas as pl
from jax.experimental.pallas import tpu as pltpu

N_Y = 4
PHASES = int(os.environ.get("K_PHASES", "4"))


def kernel(x):
    m_per, n = x.shape
    n_out = n // N_Y
    m_out = m_per * N_Y
    m_q = m_per // 4

    def body(
        x_ref, out_ref,
        comm_y, comm_dn, comm_up, xin_t, xin_b,
        ys, yr, zs_dn, zr_dn, zs_up, zr_up, xt_s, xt_r, xb_s, xb_r,
    ):
        mx = lax.axis_index("x")
        my = lax.axis_index("y")
        mz = lax.axis_index("z")
        px = 1 - mx

        def start(src_buf, dst_buf, slot, ssem, rsem, dev):
            pltpu.make_async_remote_copy(
                src_ref=src_buf.at[slot], dst_ref=dst_buf.at[slot],
                send_sem=ssem.at[slot], recv_sem=rsem.at[slot],
                device_id=dev, device_id_type=pl.DeviceIdType.MESH,
            ).start()

        def wait_r(buf, slot, rsem):
            pltpu.make_async_remote_copy(
                src_ref=buf.at[slot], dst_ref=buf.at[slot],
                send_sem=rsem.at[slot], recv_sem=rsem.at[slot],
                device_id=(mx, my, mz),
                device_id_type=pl.DeviceIdType.MESH,
            ).wait_recv()

        def wait_s(buf, slot, ssem):
            pltpu.make_async_remote_copy(
                src_ref=buf.at[slot], dst_ref=buf.at[slot],
                send_sem=ssem.at[slot], recv_sem=ssem.at[slot],
                device_id=(mx, my, mz),
                device_id_type=pl.DeviceIdType.MESH,
            ).wait_send()

        def base_of(d):
            return lax.rem(my - d + N_Y, N_Y) * m_per

        barrier_sem = pltpu.get_barrier_semaphore()

        def signal(dev):
            pl.semaphore_signal(
                barrier_sem, inc=1, device_id=dev,
                device_id_type=pl.DeviceIdType.MESH,
            )

        for zv, nbr in ((0, 1), (3, 2)):
            @pl.when(mz == zv)
            def _(zv=zv, nbr=nbr):
                cnt = 0
                if PHASES >= 2:
                    for d in range(1, N_Y):
                        signal((mx, lax.rem(my + d, N_Y), mz))
                    cnt += 3
                if PHASES >= 3:
                    signal((mx, my, nbr))
                    cnt += 1
                if PHASES >= 4:
                    signal((px, my, mz))
                    cnt += 1
                if cnt:
                    pl.semaphore_wait(barrier_sem, cnt)

        for zv in (1, 2):
            @pl.when(mz == zv)
            def _(zv=zv):
                cnt = 0
                if PHASES >= 3:
                    signal((mx, my, zv - 1))
                    signal((mx, my, zv + 1))
                    cnt += 2
                if PHASES >= 4:
                    signal((px, my, mz))
                    cnt += 1
                if cnt:
                    pl.semaphore_wait(barrier_sem, cnt)

        def local_copy():
            out_ref[pl.ds(my * m_per, m_per), :] = (
                x_ref[:, pl.ds(my * n_out, n_out)]
            )

        def drain_x_and_place(top_buf, bot_buf):
            if PHASES >= 3:
                for d in range(1, N_Y):
                    b = base_of(d)
                    out_ref[pl.ds(b + mx * m_q, m_q), :] = top_buf[d - 1]
                    out_ref[pl.ds(b + 2 * m_q + mx * m_q, m_q), :] = (
                        bot_buf[d - 1]
                    )
            if PHASES >= 4:
                for d in range(1, N_Y):
                    b = base_of(d)
                    wait_r(xin_t, d - 1, xt_r)
                    out_ref[pl.ds(b + px * m_q, m_q), :] = xin_t[d - 1]
                    wait_r(xin_b, d - 1, xb_r)
                    out_ref[pl.ds(b + 2 * m_q + px * m_q, m_q), :] = (
                        xin_b[d - 1]
                    )

        def drain_sends(sem_list):
            for buf, sems in sem_list:
                for d in range(1, N_Y):
                    wait_s(buf, d - 1, sems)

        def edge_role(zv, nbr, voff, chain_dst, chain_ssem, inbuf, in_rsem):
            if PHASES >= 2:
                for d in range(1, N_Y):
                    tgt = lax.rem(my + d, N_Y)
                    pltpu.make_async_remote_copy(
                        src_ref=x_ref.at[
                            pl.ds(voff + mx * m_q, m_q),
                            pl.ds(tgt * n_out, n_out),
                        ],
                        dst_ref=comm_y.at[d - 1],
                        send_sem=ys.at[d - 1],
                        recv_sem=yr.at[d - 1],
                        device_id=(mx, tgt, mz),
                        device_id_type=pl.DeviceIdType.MESH,
                    ).start()
            local_copy()
            my_t = comm_y if voff == 0 else inbuf
            my_b = comm_y if voff != 0 else inbuf
            if PHASES >= 2:
                for d in range(1, N_Y):
                    wait_r(comm_y, d - 1, yr)
                    if PHASES >= 3:
                        start(comm_y, chain_dst, d - 1, chain_ssem,
                              zr_dn if voff == 0 else zr_up, (mx, my, nbr))
                    if PHASES >= 4:
                        start(comm_y, xin_t if voff == 0 else xin_b, d - 1,
                              xt_s if voff == 0 else xb_s,
                              xt_r if voff == 0 else xb_r, (px, my, mz))
            if PHASES >= 3:
                for d in range(1, N_Y):
                    wait_r(inbuf, d - 1, in_rsem)
                    if PHASES >= 4:
                        start(inbuf, xin_b if voff == 0 else xin_t, d - 1,
                              xb_s if voff == 0 else xt_s,
                              xb_r if voff == 0 else xt_r, (px, my, mz))
            drain_x_and_place(my_t, my_b)
            drains = []
            if PHASES >= 2:
                drains.append((comm_y, ys))
            if PHASES >= 3:
                drains.append((comm_y, chain_ssem))
            if PHASES >= 4:
                drains.append((comm_y, xt_s))
                drains.append((inbuf, xb_s))
            drain_sends(drains)

        def interior_role(zv):
            local_copy()
            if PHASES >= 3:
                for d in range(1, N_Y):
                    wait_r(comm_dn, d - 1, zr_dn)
                    start(comm_dn, comm_dn, d - 1, zs_dn, zr_dn,
                          (mx, my, zv + 1))
                    if PHASES >= 4:
                        start(comm_dn, xin_t, d - 1, xt_s, xt_r, (px, my, mz))
                for d in range(1, N_Y):
                    wait_r(comm_up, d - 1, zr_up)
                    start(comm_up, comm_up, d - 1, zs_up, zr_up,
                          (mx, my, zv - 1))
                    if PHASES >= 4:
                        start(comm_up, xin_b, d - 1, xb_s, xb_r, (px, my, mz))
            drain_x_and_place(comm_dn, comm_up)
            drains = []
            if PHASES >= 3:
                drains.append((comm_dn, zs_dn))
                drains.append((comm_up, zs_up))
            if PHASES >= 4:
                drains.append((comm_dn, xt_s))
                drains.append((comm_up, xb_s))
            drain_sends(drains)

        @pl.when(mz == 0)
        def _():
            edge_role(0, 1, 0, comm_dn, zs_dn, comm_up, zr_up)

        @pl.when(mz == 3)
        def _():
            edge_role(3, 2, 2 * m_q, comm_up, zs_up, comm_dn, zr_dn)

        @pl.when(mz == 1)
        def _():
            interior_role(1)

        @pl.when(mz == 2)
        def _():
            interior_role(2)

    out_shape = jax.ShapeDtypeStruct((m_out, n_out), x.dtype)
    sem3 = pltpu.SemaphoreType.DMA((N_Y - 1,))
    buf3 = pltpu.VMEM((N_Y - 1, m_q, n_out), x.dtype)
    return pl.pallas_call(
        body,
        out_shape=out_shape,
        in_specs=[pl.BlockSpec(memory_space=pltpu.VMEM)],
        out_specs=pl.BlockSpec(memory_space=pltpu.VMEM),
        scratch_shapes=[
            buf3,
            buf3,
            buf3,
            buf3,
            buf3,
            sem3, sem3,
            sem3, sem3,
            sem3, sem3,
            sem3, sem3,
            sem3, sem3,
        ],
        compiler_params=pltpu.CompilerParams(collective_id=0),
    )(x)


# device time: 65570 ns/iter; 1.2762x vs baseline; 1.2762x over previous
import jax
import jax.numpy as jnp
from jax import lax
from jax.experimental import pallas as pl
from jax.experimental.pallas import tpu as pltpu

N_Y = 4
N_Q = 2


def kernel(x):
    m_per, n = x.shape
    n_out = n // N_Y
    m_out = m_per * N_Y
    m_half = m_per // 2
    m_q = m_half // N_Q
    n_slots = (N_Y - 1) * N_Q

    def body(x_ref, out_ref, comm_y, comm_x, ys, yr, xs, xr):
        mx = lax.axis_index("x")
        my = lax.axis_index("y")
        mz = lax.axis_index("z")
        px = 1 - mx

        barrier_sem = pltpu.get_barrier_semaphore()
        for d in range(1, N_Y):
            pl.semaphore_signal(
                barrier_sem, inc=1,
                device_id=(mx, lax.rem(my + d, N_Y), mz),
                device_id_type=pl.DeviceIdType.MESH,
            )
        pl.semaphore_signal(
            barrier_sem, inc=1, device_id=(px, my, mz),
            device_id_type=pl.DeviceIdType.MESH,
        )
        pl.semaphore_wait(barrier_sem, N_Y)

        for d in range(1, N_Y):
            tgt = lax.rem(my + d, N_Y)
            for q in range(N_Q):
                k = (d - 1) * N_Q + q
                pltpu.make_async_remote_copy(
                    src_ref=x_ref.at[
                        pl.ds(mx * m_half + q * m_q, m_q),
                        pl.ds(tgt * n_out, n_out),
                    ],
                    dst_ref=comm_y.at[k],
                    send_sem=ys.at[k],
                    recv_sem=yr.at[k],
                    device_id=(mx, tgt, mz),
                    device_id_type=pl.DeviceIdType.MESH,
                ).start()

        out_ref[pl.ds(my * m_per, m_per), :] = (
            x_ref[:, pl.ds(my * n_out, n_out)]
        )

        for d in range(1, N_Y):
            base = lax.rem(my - d + N_Y, N_Y) * m_per
            for q in range(N_Q):
                k = (d - 1) * N_Q + q
                row0 = base + mx * m_half + q * m_q
                pltpu.make_async_remote_copy(
                    src_ref=comm_y.at[k], dst_ref=comm_y.at[k],
                    send_sem=yr.at[k], recv_sem=yr.at[k],
                    device_id=(mx, my, mz),
                    device_id_type=pl.DeviceIdType.MESH,
                ).wait_recv()
                pltpu.make_async_remote_copy(
                    src_ref=comm_y.at[k],
                    dst_ref=comm_x.at[k],
                    send_sem=xs.at[k],
                    recv_sem=xr.at[k],
                    device_id=(px, my, mz),
                    device_id_type=pl.DeviceIdType.MESH,
                ).start()
                out_ref[pl.ds(row0, m_q), :] = comm_y[k]

        for d in range(1, N_Y):
            base = lax.rem(my - d + N_Y, N_Y) * m_per
            for q in range(N_Q):
                k = (d - 1) * N_Q + q
                row0 = base + px * m_half + q * m_q
                pltpu.make_async_remote_copy(
                    src_ref=comm_x.at[k],
                    dst_ref=comm_x.at[k],
                    send_sem=xr.at[k],
                    recv_sem=xr.at[k],
                    device_id=(px, my, mz),
                    device_id_type=pl.DeviceIdType.MESH,
                ).wait_recv()
                out_ref[pl.ds(row0, m_q), :] = comm_x[k]

        for sems in (ys, xs):
            for k in range(n_slots):
                pltpu.make_async_remote_copy(
                    src_ref=comm_y.at[k], dst_ref=comm_y.at[k],
                    send_sem=sems.at[k], recv_sem=sems.at[k],
                    device_id=(mx, my, mz),
                    device_id_type=pl.DeviceIdType.MESH,
                ).wait_send()

    out_shape = jax.ShapeDtypeStruct((m_out, n_out), x.dtype)
    return pl.pallas_call(
        body,
        out_shape=out_shape,
        in_specs=[pl.BlockSpec(memory_space=pltpu.VMEM)],
        out_specs=pl.BlockSpec(memory_space=pltpu.VMEM),
        scratch_shapes=[
            pltpu.VMEM((n_slots, m_q, n_out), x.dtype),
            pltpu.VMEM((n_slots, m_q, n_out), x.dtype),
            pltpu.SemaphoreType.DMA((n_slots,)),
            pltpu.SemaphoreType.DMA((n_slots,)),
            pltpu.SemaphoreType.DMA((n_slots,)),
            pltpu.SemaphoreType.DMA((n_slots,)),
        ],
        compiler_params=pltpu.CompilerParams(collective_id=0),
    )(x)


# device time: 52838 ns/iter; 1.5838x vs baseline; 1.2410x over previous
import jax
import jax.numpy as jnp
from jax import lax
from jax.experimental import pallas as pl
from jax.experimental.pallas import tpu as pltpu

N_Y = 4


def kernel(x):
    m_per, n = x.shape
    n_out = n // N_Y
    m_out = m_per * N_Y
    m_q = m_per // 4
    m_h = m_q // 2

    def body(
        x_ref, out_ref,
        comm_y, zin, xin, zdh, xdh,
        ys, yr, zs1, zr1, xs1, xr1, zs2, zr2, xs2, xr2,
    ):
        mx = lax.axis_index("x")
        my = lax.axis_index("y")
        mz = lax.axis_index("z")
        px = 1 - mx
        zb = lax.rem(mz, 2)
        pz = mz + 1 - 2 * zb

        barrier_sem = pltpu.get_barrier_semaphore()
        for d in range(1, N_Y):
            pl.semaphore_signal(
                barrier_sem, inc=1,
                device_id=(mx, lax.rem(my + d, N_Y), mz),
                device_id_type=pl.DeviceIdType.MESH,
            )
        for dev in ((mx, my, pz), (px, my, mz)):
            pl.semaphore_signal(
                barrier_sem, inc=1, device_id=dev,
                device_id_type=pl.DeviceIdType.MESH,
            )
        pl.semaphore_wait(barrier_sem, N_Y + 1)

        for d in range(1, N_Y):
            tgt = lax.rem(my + d, N_Y)
            pltpu.make_async_remote_copy(
                src_ref=x_ref.at[
                    pl.ds(mx * 2 * m_q + zb * m_q, m_q),
                    pl.ds(tgt * n_out, n_out),
                ],
                dst_ref=comm_y.at[d - 1],
                send_sem=ys.at[d - 1],
                recv_sem=yr.at[d - 1],
                device_id=(mx, tgt, mz),
                device_id_type=pl.DeviceIdType.MESH,
            ).start()

        out_ref[pl.ds(my * m_per, m_per), :] = (
            x_ref[:, pl.ds(my * n_out, n_out)]
        )

        def wait_recv(buf, slot, rsem):
            pltpu.make_async_remote_copy(
                src_ref=buf.at[slot], dst_ref=buf.at[slot],
                send_sem=rsem.at[slot], recv_sem=rsem.at[slot],
                device_id=(mx, my, mz),
                device_id_type=pl.DeviceIdType.MESH,
            ).wait_recv()

        def send(src, dst_buf, slot, ssem, rsem, dev):
            pltpu.make_async_remote_copy(
                src_ref=src, dst_ref=dst_buf.at[slot],
                send_sem=ssem.at[slot], recv_sem=rsem.at[slot],
                device_id=dev, device_id_type=pl.DeviceIdType.MESH,
            ).start()

        def rows(base, xbit, zbit):
            return base + xbit * 2 * m_q + zbit * m_q

        for d in range(1, N_Y):
            k = d - 1
            base = lax.rem(my - d + N_Y, N_Y) * m_per
            wait_recv(comm_y, k, yr)
            send(comm_y.at[k], zin, k, zs1, zr1, (mx, my, pz))
            send(comm_y.at[k], xin, k, xs1, xr1, (px, my, mz))
            out_ref[pl.ds(rows(base, mx, zb), m_q), :] = comm_y[k]

        for d in range(1, N_Y):
            k = d - 1
            base = lax.rem(my - d + N_Y, N_Y) * m_per
            wait_recv(zin, k, zr1)
            send(zin.at[k, pl.ds(m_h, m_h), :], xdh, k, xs2, xr2,
                 (px, my, mz))
            out_ref[pl.ds(rows(base, mx, 1 - zb), m_q), :] = zin[k]

        for d in range(1, N_Y):
            k = d - 1
            base = lax.rem(my - d + N_Y, N_Y) * m_per
            wait_recv(xin, k, xr1)
            send(xin.at[k, pl.ds(0, m_h), :], zdh, k, zs2, zr2,
                 (mx, my, pz))
            out_ref[pl.ds(rows(base, px, zb), m_q), :] = xin[k]

        for d in range(1, N_Y):
            k = d - 1
            base = lax.rem(my - d + N_Y, N_Y) * m_per
            r0 = rows(base, px, 1 - zb)
            wait_recv(zdh, k, zr2)
            out_ref[pl.ds(r0, m_h), :] = zdh[k]
            wait_recv(xdh, k, xr2)
            out_ref[pl.ds(r0 + m_h, m_h), :] = xdh[k]

        for sems in (ys, zs1, xs1):
            for k in range(N_Y - 1):
                pltpu.make_async_remote_copy(
                    src_ref=comm_y.at[k], dst_ref=comm_y.at[k],
                    send_sem=sems.at[k], recv_sem=sems.at[k],
                    device_id=(mx, my, mz),
                    device_id_type=pl.DeviceIdType.MESH,
                ).wait_send()
        for sems in (zs2, xs2):
            for k in range(N_Y - 1):
                pltpu.make_async_remote_copy(
                    src_ref=zdh.at[k], dst_ref=zdh.at[k],
                    send_sem=sems.at[k], recv_sem=sems.at[k],
                    device_id=(mx, my, mz),
                    device_id_type=pl.DeviceIdType.MESH,
                ).wait_send()

    out_shape = jax.ShapeDtypeStruct((m_out, n_out), x.dtype)
    semq = pltpu.SemaphoreType.DMA((N_Y - 1,))
    bufq = pltpu.VMEM((N_Y - 1, m_q, n_out), x.dtype)
    bufh = pltpu.VMEM((N_Y - 1, m_h, n_out), x.dtype)
    return pl.pallas_call(
        body,
        out_shape=out_shape,
        in_specs=[pl.BlockSpec(memory_space=pltpu.VMEM)],
        out_specs=pl.BlockSpec(memory_space=pltpu.VMEM),
        scratch_shapes=[
            bufq,
            bufq,
            bufq,
            bufh,
            bufh,
            semq, semq,
            semq, semq,
            semq, semq,
            semq, semq,
            semq, semq,
        ],
        compiler_params=pltpu.CompilerParams(collective_id=0),
    )(x)


# device time: 52274 ns/iter; 1.6009x vs baseline; 1.0108x over previous
import jax
import jax.numpy as jnp
from jax import lax
from jax.experimental import pallas as pl
from jax.experimental.pallas import tpu as pltpu

N_Y = 4


def kernel(x):
    m_per, n = x.shape
    n_out = n // N_Y
    m_out = m_per * N_Y
    m_q = m_per // 4
    m_h = m_q // 2

    def body(
        x_ref, out_ref,
        comm_y, zin, xin, ddh,
        ys, yr, zs1, zr1, xs1, xr1, zs2, zr2, xs2, xr2,
    ):
        mx = lax.axis_index("x")
        my = lax.axis_index("y")
        mz = lax.axis_index("z")
        px = 1 - mx
        zb = lax.rem(mz, 2)
        pz = mz + 1 - 2 * zb

        barrier_sem = pltpu.get_barrier_semaphore()
        for d in range(1, N_Y):
            pl.semaphore_signal(
                barrier_sem, inc=1,
                device_id=(mx, lax.rem(my + d, N_Y), mz),
                device_id_type=pl.DeviceIdType.MESH,
            )
        for dev in ((mx, my, pz), (px, my, mz)):
            pl.semaphore_signal(
                barrier_sem, inc=1, device_id=dev,
                device_id_type=pl.DeviceIdType.MESH,
            )
        pl.semaphore_wait(barrier_sem, N_Y + 1)

        for d in range(1, N_Y):
            tgt = lax.rem(my + d, N_Y)
            pltpu.make_async_remote_copy(
                src_ref=x_ref.at[
                    pl.ds(mx * 2 * m_q + zb * m_q, m_q),
                    pl.ds(tgt * n_out, n_out),
                ],
                dst_ref=comm_y.at[d - 1],
                send_sem=ys.at[d - 1],
                recv_sem=yr.at[d - 1],
                device_id=(mx, tgt, mz),
                device_id_type=pl.DeviceIdType.MESH,
            ).start()

        out_ref[pl.ds(my * m_per, m_per), :] = (
            x_ref[:, pl.ds(my * n_out, n_out)]
        )

        def wait_recv(buf, slot, rsem):
            pltpu.make_async_remote_copy(
                src_ref=buf.at[slot], dst_ref=buf.at[slot],
                send_sem=rsem.at[slot], recv_sem=rsem.at[slot],
                device_id=(mx, my, mz),
                device_id_type=pl.DeviceIdType.MESH,
            ).wait_recv()

        def wait_half(buf, slot, rsem):
            pltpu.make_async_remote_copy(
                src_ref=buf.at[slot, pl.ds(0, m_h), :],
                dst_ref=buf.at[slot, pl.ds(0, m_h), :],
                send_sem=rsem.at[slot], recv_sem=rsem.at[slot],
                device_id=(mx, my, mz),
                device_id_type=pl.DeviceIdType.MESH,
            ).wait_recv()

        def send(src_ref, dst_ref, ssem, rsem, slot, dev):
            pltpu.make_async_remote_copy(
                src_ref=src_ref, dst_ref=dst_ref,
                send_sem=ssem.at[slot], recv_sem=rsem.at[slot],
                device_id=dev, device_id_type=pl.DeviceIdType.MESH,
            ).start()

        def rows(base, xbit, zbit):
            return base + xbit * 2 * m_q + zbit * m_q

        for d in range(1, N_Y):
            k = d - 1
            base = lax.rem(my - d + N_Y, N_Y) * m_per
            wait_recv(comm_y, k, yr)
            send(comm_y.at[k], zin.at[k], zs1, zr1, k, (mx, my, pz))
            send(comm_y.at[k], xin.at[k], xs1, xr1, k, (px, my, mz))
            out_ref[pl.ds(rows(base, mx, zb), m_q), :] = comm_y[k]
            wait_recv(zin, k, zr1)
            send(zin.at[k, pl.ds(m_h, m_h), :], ddh.at[k, pl.ds(m_h, m_h), :],
                 xs2, xr2, k, (px, my, mz))
            out_ref[pl.ds(rows(base, mx, 1 - zb), m_q), :] = zin[k]
            wait_recv(xin, k, xr1)
            send(xin.at[k, pl.ds(0, m_h), :], ddh.at[k, pl.ds(0, m_h), :],
                 zs2, zr2, k, (mx, my, pz))
            out_ref[pl.ds(rows(base, px, zb), m_q), :] = xin[k]

        for d in range(1, N_Y):
            k = d - 1
            base = lax.rem(my - d + N_Y, N_Y) * m_per
            wait_half(ddh, k, zr2)
            wait_half(ddh, k, xr2)
            out_ref[pl.ds(rows(base, px, 1 - zb), m_q), :] = ddh[k]

        for sems in (ys, zs1, xs1):
            for k in range(N_Y - 1):
                pltpu.make_async_remote_copy(
                    src_ref=comm_y.at[k], dst_ref=comm_y.at[k],
                    send_sem=sems.at[k], recv_sem=sems.at[k],
                    device_id=(mx, my, mz),
                    device_id_type=pl.DeviceIdType.MESH,
                ).wait_send()
        for sems in (zs2, xs2):
            for k in range(N_Y - 1):
                pltpu.make_async_remote_copy(
                    src_ref=ddh.at[k, pl.ds(0, m_h), :],
                    dst_ref=ddh.at[k, pl.ds(0, m_h), :],
                    send_sem=sems.at[k], recv_sem=sems.at[k],
                    device_id=(mx, my, mz),
                    device_id_type=pl.DeviceIdType.MESH,
                ).wait_send()

    out_shape = jax.ShapeDtypeStruct((m_out, n_out), x.dtype)
    semq = pltpu.SemaphoreType.DMA((N_Y - 1,))
    bufq = pltpu.VMEM((N_Y - 1, m_q, n_out), x.dtype)
    bufh = pltpu.VMEM((N_Y - 1, m_h, n_out), x.dtype)
    return pl.pallas_call(
        body,
        out_shape=out_shape,
        in_specs=[pl.BlockSpec(memory_space=pltpu.VMEM)],
        out_specs=pl.BlockSpec(memory_space=pltpu.VMEM),
        scratch_shapes=[
            bufq,
            bufq,
            bufq,
            bufq,
            semq, semq,
            semq, semq,
            semq, semq,
            semq, semq,
            semq, semq,
        ],
        compiler_params=pltpu.CompilerParams(collective_id=0),
    )(x)
